# Initial kernel scaffold; baseline (speedup 1.0000x reference)
import jax
import jax.numpy as jnp
from jax.experimental import pallas as pl


def kernel(x_pfas_sites, x_gw_wells, x_sw_stations, ei_ps_gw, ei_gw_ps, ei_ps_sw, ei_sw_ps, ei_sw_gw, ei_gw_sw, ei_gw_gw, ei_gw_self, ei_sw_self, ei_ps_self, params):
    raise NotImplementedError("write your pallas kernel here")


# speedup vs baseline: 2.3695x; 2.3695x over previous
import functools

import jax
import jax.numpy as jnp
from jax import lax
from jax.experimental import pallas as pl
from jax.experimental.pallas import tpu as pltpu
from jax.experimental.pallas import tpu_sc as plsc

N = 10000
D = 128
E = 80000
NC = 2
NS = 16
_CHUNK = 128

_REL_DEFS = [
    ('ps_gw', 'ps', 'gw', E),
    ('gw_ps', 'gw', 'ps', E),
    ('ps_sw', 'ps', 'sw', E),
    ('sw_ps', 'sw', 'ps', E),
    ('sw_gw', 'sw', 'gw', E),
    ('gw_sw', 'gw', 'sw', E),
    ('gw_gw', 'gw', 'gw', E),
    ('gw_self', 'gw', 'gw', N),
    ('sw_self', 'sw', 'sw', N),
    ('ps_self', 'ps', 'ps', N),
]
_NREL = len(_REL_DEFS)
_RIDX = {r[0]: i for i, r in enumerate(_REL_DEFS)}
_TYPES = ('ps', 'gw', 'sw')

_SLOTS = [
    ('ps_gw', 'sw_ps'),
    ('gw_ps', 'sw_gw'),
    ('ps_sw', 'gw_sw'),
    ('gw_self', 'gw_gw'),
    ('sw_self', None),
    ('ps_self', None),
]

_ROWS_MAIN = 624
_ROWS_LAST = N - (NS - 1) * _ROWS_MAIN


def _padded(n_edges):
    q = NS * _CHUNK
    return ((n_edges + q - 1) // q) * q


def _build_agg(with_counts):
    out_type = [jax.ShapeDtypeStruct((N, D), jnp.float32)
                for _ in range(_NREL)]
    if with_counts:
        out_type += [jax.ShapeDtypeStruct((N, D), jnp.float32)
                     for _ in range(_NREL)]
    mesh = plsc.VectorSubcoreMesh(core_axis_name="c", subcore_axis_name="s")
    scratch = [
        pltpu.VMEM((_CHUNK,), jnp.int32),
        pltpu.VMEM((_CHUNK,), jnp.int32),
        pltpu.VMEM((_CHUNK, D), jnp.float32),
        pltpu.VMEM((_CHUNK, D), jnp.float32),
        pltpu.VMEM_SHARED((N + 8, D), jnp.float32),
        pltpu.SemaphoreType.DMA,
    ]

    @functools.partial(pl.kernel, out_type=out_type, mesh=mesh,
                       scratch_types=scratch)
    def agg(*refs):
        k = 0
        x_refs = {t: refs[k + i] for i, t in enumerate(_TYPES)}
        k += 3
        z2_ref = refs[k]; k += 1
        ones_ref = refs[k]; k += 1
        src_refs = refs[k:k + _NREL]; k += _NREL
        dst_refs = refs[k:k + _NREL]; k += _NREL
        seg_refs = refs[k:k + _NREL]; k += _NREL
        if with_counts:
            cnt_refs = refs[k:k + _NREL]; k += _NREL
        idx_s, idx_d, rows, ones_v, acc, sem = refs[k:]

        cid = lax.axis_index("c")
        sid = lax.axis_index("s")
        row0 = sid * _ROWS_MAIN

        def zero_slices():
            @pl.when(sid < NS - 1)
            def _():
                pltpu.sync_copy(z2_ref.at[pl.ds(row0, _ROWS_MAIN)],
                                acc.at[pl.ds(row0, _ROWS_MAIN)])

            @pl.when(sid == NS - 1)
            def _():
                o = (NS - 1) * _ROWS_MAIN
                pltpu.sync_copy(z2_ref.at[pl.ds(o, _ROWS_LAST)],
                                acc.at[pl.ds(o, _ROWS_LAST)])

        def edge_loop(ri):
            n_pad = _padded(_REL_DEFS[ri][3])
            iters = n_pad // (NS * _CHUNK)
            x_ref = x_refs[_REL_DEFS[ri][1]]

            def body(it, carry):
                base = (sid + it * NS) * _CHUNK
                pltpu.sync_copy(src_refs[ri].at[pl.ds(base, _CHUNK)], idx_s)
                pltpu.sync_copy(dst_refs[ri].at[pl.ds(base, _CHUNK)], idx_d)
                pltpu.async_copy(x_ref.at[idx_s], rows, sem).wait()
                pltpu.sync_copy(rows, acc.at[idx_d], add=True)
                return carry

            lax.fori_loop(0, iters, body, 0)

        def count_loop(ri):
            n_pad = _padded(_REL_DEFS[ri][3])
            iters = n_pad // (NS * _CHUNK)

            def body(it, carry):
                base = (sid + it * NS) * _CHUNK
                pltpu.sync_copy(dst_refs[ri].at[pl.ds(base, _CHUNK)], idx_d)
                pltpu.sync_copy(ones_v, acc.at[idx_d], add=True)
                return carry

            lax.fori_loop(0, iters, body, 0)

        def writeback(ri, refs_list):
            @pl.when(sid < NS - 1)
            def _():
                pltpu.sync_copy(acc.at[pl.ds(row0, _ROWS_MAIN)],
                                refs_list[ri].at[pl.ds(row0, _ROWS_MAIN)])

            @pl.when(sid == NS - 1)
            def _():
                o = (NS - 1) * _ROWS_MAIN
                pltpu.sync_copy(acc.at[pl.ds(o, _ROWS_LAST)],
                                refs_list[ri].at[pl.ds(o, _ROWS_LAST)])

        def run_slots(loop_fn, out_list):
            for rel0, rel1 in _SLOTS:
                zero_slices()
                plsc.subcore_barrier()
                for c, rel in ((0, rel0), (1, rel1)):
                    if rel is not None:
                        @pl.when(cid == c)
                        def _(ri=_RIDX[rel]):
                            loop_fn(ri)
                plsc.subcore_barrier()
                for c, rel in ((0, rel0), (1, rel1)):
                    if rel is not None:
                        @pl.when(cid == c)
                        def _(ri=_RIDX[rel]):
                            writeback(ri, out_list)

        run_slots(edge_loop, seg_refs)
        if with_counts:
            pltpu.sync_copy(ones_ref, ones_v)
            run_slots(count_loop, cnt_refs)

    return agg


_AGG_L1 = _build_agg(with_counts=True)
_AGG_L2 = _build_agg(with_counts=False)

_TCB = 1000


def _build_tc(n_rel, final):
    grid = (N // _TCB,)
    in_specs = [pl.BlockSpec((_TCB, D), lambda i: (i, 0))]
    for _ in range(n_rel):
        in_specs += [pl.BlockSpec((_TCB, D), lambda i: (i, 0)),
                     pl.BlockSpec((_TCB, 1), lambda i: (i, 0)),
                     pl.BlockSpec((D, D), lambda i: (0, 0))]
    in_specs += [pl.BlockSpec((D, D), lambda i: (0, 0)),
                 pl.BlockSpec((1, D), lambda i: (0, 0))]
    if final:
        in_specs += [pl.BlockSpec((D, 1), lambda i: (0, 0)),
                     pl.BlockSpec((1, 1), lambda i: (0, 0)),
                     pl.BlockSpec((1, 1), lambda i: (0, 0))]
        out_spec = pl.BlockSpec((_TCB, 1), lambda i: (i, 0))
        out_shape = jax.ShapeDtypeStruct((N, 1), jnp.float32)
    else:
        out_spec = pl.BlockSpec((_TCB, D), lambda i: (i, 0))
        out_shape = jax.ShapeDtypeStruct((N, D), jnp.float32)

    def body(*refs):
        x_ref = refs[0]
        k = 1
        acc = None
        for _ in range(n_rel):
            seg_ref, cnt_ref, w_ref = refs[k], refs[k + 1], refs[k + 2]
            k += 3
            recip = 1.0 / jnp.maximum(cnt_ref[...], 1.0)
            m = jnp.dot(seg_ref[...] * recip, w_ref[...],
                        preferred_element_type=jnp.float32)
            acc = m if acc is None else acc + m
        wr_ref, b_ref = refs[k], refs[k + 1]
        k += 2
        acc = acc + jnp.dot(x_ref[...], wr_ref[...],
                            preferred_element_type=jnp.float32) + b_ref[...]
        acc = jnp.maximum(acc, 0.0)
        if final:
            lw_ref, lb_ref, a_ref = refs[k], refs[k + 1], refs[k + 2]
            z = jnp.dot(acc, lw_ref[...],
                        preferred_element_type=jnp.float32) + lb_ref[...]
            acc = jnp.where(z >= 0, z, a_ref[...] * z)
        refs[-1][...] = acc

    return pl.pallas_call(body, grid=grid, in_specs=in_specs,
                          out_specs=out_spec, out_shape=out_shape)


_TC_CALLS = {}


def _tc_call(n_rel, final):
    key = (n_rel, final)
    if key not in _TC_CALLS:
        _TC_CALLS[key] = _build_tc(n_rel, final)
    return _TC_CALLS[key]


def _hetero_layer(xs, segs, cnts, layer_params, final_head=None):
    out = {}
    for d_t in _TYPES:
        rel_ids = [i for i, r in enumerate(_REL_DEFS) if r[2] == d_t]
        args = [xs[d_t]]
        wr_sum = None
        b_sum = None
        for ri in rel_ids:
            name = _REL_DEFS[ri][0]
            Wl, bl, Wr = layer_params[name]
            args += [segs[ri], cnts[ri], Wl.T]
            wr_sum = Wr if wr_sum is None else wr_sum + Wr
            b_sum = bl if b_sum is None else b_sum + bl
        args += [wr_sum.T, b_sum.reshape(1, D)]
        fin = final_head is not None and d_t in ('gw', 'sw')
        if fin:
            lin_W, lin_b, a = final_head
            args += [lin_W.T, lin_b.reshape(1, 1), a.reshape(1, 1)]
        out[d_t] = _tc_call(len(rel_ids), fin)(*args)
    return out


def kernel(x_pfas_sites, x_gw_wells, x_sw_stations,
           ei_ps_gw, ei_gw_ps, ei_ps_sw, ei_sw_ps, ei_sw_gw, ei_gw_sw,
           ei_gw_gw, ei_gw_self, ei_sw_self, ei_ps_self, params):
    xs = {'ps': x_pfas_sites, 'gw': x_gw_wells, 'sw': x_sw_stations}
    eis = {'ps_gw': ei_ps_gw, 'gw_ps': ei_gw_ps, 'ps_sw': ei_ps_sw,
           'sw_ps': ei_sw_ps, 'sw_gw': ei_sw_gw, 'gw_sw': ei_gw_sw,
           'gw_gw': ei_gw_gw, 'gw_self': ei_gw_self, 'sw_self': ei_sw_self,
           'ps_self': ei_ps_self}

    srcs, dsts = [], []
    for r in _REL_DEFS:
        s = eis[r[0]][0].astype(jnp.int32)
        d = eis[r[0]][1].astype(jnp.int32)
        pad = _padded(r[3]) - r[3]
        if pad:
            s = jnp.concatenate([s, jnp.zeros((pad,), jnp.int32)])
            d = jnp.concatenate([d, jnp.full((pad,), N, jnp.int32)])
        srcs.append(s)
        dsts.append(d)
    zeros2d = jnp.zeros((N, D), jnp.float32)
    ones2d = jnp.ones((_CHUNK, D), jnp.float32)

    out1 = _AGG_L1(xs['ps'], xs['gw'], xs['sw'], zeros2d, ones2d,
                   *srcs, *dsts)
    segs1 = list(out1[:_NREL])
    cnts = [c[:, :1] for c in out1[_NREL:]]

    h1 = _hetero_layer(xs, segs1, cnts, params['conv1'])

    out2 = _AGG_L2(h1['ps'], h1['gw'], h1['sw'], zeros2d, ones2d,
                   *srcs, *dsts)
    segs2 = list(out2)

    head = (params['lin_W'], params['lin_b'], params['prelu_a'])
    h2 = _hetero_layer(h1, segs2, cnts, params['conv2'], final_head=head)

    return (h2['gw'], h2['sw'], h2['ps'])


# speedup vs baseline: 2.5620x; 1.0813x over previous
import functools

import jax
import jax.numpy as jnp
from jax import lax
from jax.experimental import pallas as pl
from jax.experimental.pallas import tpu as pltpu
from jax.experimental.pallas import tpu_sc as plsc

N = 10000
D = 128
E = 80000
NC = 2
NS = 16
_CHUNK = 128

_REL_DEFS = [
    ('ps_gw', 'ps', 'gw', E),
    ('gw_ps', 'gw', 'ps', E),
    ('ps_sw', 'ps', 'sw', E),
    ('sw_ps', 'sw', 'ps', E),
    ('sw_gw', 'sw', 'gw', E),
    ('gw_sw', 'gw', 'sw', E),
    ('gw_gw', 'gw', 'gw', E),
    ('gw_self', 'gw', 'gw', N),
    ('sw_self', 'sw', 'sw', N),
    ('ps_self', 'ps', 'ps', N),
]
_NREL = len(_REL_DEFS)
_RIDX = {r[0]: i for i, r in enumerate(_REL_DEFS)}
_TYPES = ('ps', 'gw', 'sw')

_SLOTS = [
    ('ps_gw', 'sw_ps'),
    ('gw_ps', 'sw_gw'),
    ('ps_sw', 'gw_sw'),
    ('gw_self', 'gw_gw'),
    ('sw_self', None),
    ('ps_self', None),
]

_ROWS_MAIN = 624
_ROWS_LAST = N - (NS - 1) * _ROWS_MAIN


def _padded(n_edges):
    q = NS * _CHUNK
    return ((n_edges + q - 1) // q) * q


_OVERSHOOT = 2 * NS * _CHUNK


_CORE_OF = {}
for _r0, _r1 in _SLOTS:
    if _r0 is not None:
        _CORE_OF[_r0] = 0
    if _r1 is not None:
        _CORE_OF[_r1] = 1


def _build_agg(with_counts):
    out_type = [jax.ShapeDtypeStruct((N, D), jnp.float32)
                for _ in range(_NREL)]
    if with_counts:
        out_type += [jax.ShapeDtypeStruct((N, D), jnp.float32)
                     for _ in range(NC)]
    mesh = plsc.VectorSubcoreMesh(core_axis_name="c", subcore_axis_name="s")
    scratch = [
        pltpu.VMEM((2, _CHUNK), jnp.int32),
        pltpu.VMEM((2, _CHUNK), jnp.int32),
        pltpu.VMEM((_CHUNK, D), jnp.float32),
        pltpu.VMEM((_CHUNK, D), jnp.float32),
        pltpu.VMEM_SHARED((N + 8, D), jnp.float32),
        pltpu.SemaphoreType.DMA,
        pltpu.SemaphoreType.DMA,
        pltpu.SemaphoreType.DMA,
    ]

    @functools.partial(pl.kernel, out_type=out_type, mesh=mesh,
                       scratch_types=scratch)
    def agg(*refs):
        k = 0
        x_refs = {t: refs[k + i] for i, t in enumerate(_TYPES)}
        k += 3
        z2_ref = refs[k]; k += 1
        oh_ref = refs[k]; k += 1
        ei_refs = refs[k:k + _NREL]; k += _NREL
        seg_refs = refs[k:k + _NREL]; k += _NREL
        if with_counts:
            cnt_refs = refs[k:k + NC]; k += NC
        idx_sd0, idx_sd1, rows0, rows1, acc, sg0, si0, si1 = refs[k:]
        idx_sd = (idx_sd0, idx_sd1)
        si = (si0, si1)

        cid = lax.axis_index("c")
        sid = lax.axis_index("s")
        row0 = sid * _ROWS_MAIN

        def zero_slices():
            @pl.when(sid < NS - 1)
            def _():
                pltpu.sync_copy(z2_ref.at[pl.ds(row0, _ROWS_MAIN)],
                                acc.at[pl.ds(row0, _ROWS_MAIN)])

            @pl.when(sid == NS - 1)
            def _():
                o = (NS - 1) * _ROWS_MAIN
                pltpu.sync_copy(z2_ref.at[pl.ds(o, _ROWS_LAST)],
                                acc.at[pl.ds(o, _ROWS_LAST)])

        def edge_loop(ri):
            iters = _padded(_REL_DEFS[ri][3]) // (NS * _CHUNK)
            x_ref = x_refs[_REL_DEFS[ri][1]]

            def body(it, carry):
                base = (sid + it * NS) * _CHUNK
                pltpu.sync_copy(
                    ei_refs[ri].at[:, pl.ds(base, _CHUNK)], idx_sd0)
                pltpu.async_copy(x_ref.at[idx_sd0.at[0]], rows0,
                                 sg0).wait()
                pltpu.sync_copy(rows0, acc.at[idx_sd0.at[1]], add=True)
                return carry

            lax.fori_loop(0, iters, body, 0)

        def count_loop(ri):
            iters = _padded(_REL_DEFS[ri][3]) // (NS * _CHUNK)

            def body(it, carry):
                base = (sid + it * NS) * _CHUNK
                pltpu.sync_copy(
                    ei_refs[ri].at[:, pl.ds(base, _CHUNK)], idx_sd0)
                pltpu.sync_copy(rows1, acc.at[idx_sd0.at[1]], add=True)
                return carry

            lax.fori_loop(0, iters, body, 0)

        def writeback(ri, refs_list):
            @pl.when(sid < NS - 1)
            def _():
                pltpu.sync_copy(acc.at[pl.ds(row0, _ROWS_MAIN)],
                                refs_list[ri].at[pl.ds(row0, _ROWS_MAIN)])

            @pl.when(sid == NS - 1)
            def _():
                o = (NS - 1) * _ROWS_MAIN
                pltpu.sync_copy(acc.at[pl.ds(o, _ROWS_LAST)],
                                refs_list[ri].at[pl.ds(o, _ROWS_LAST)])

        for rel0, rel1 in _SLOTS:
            zero_slices()
            plsc.subcore_barrier()
            for c, rel in ((0, rel0), (1, rel1)):
                if rel is not None:
                    @pl.when(cid == c)
                    def _(ri=_RIDX[rel]):
                        edge_loop(ri)
            plsc.subcore_barrier()
            for c, rel in ((0, rel0), (1, rel1)):
                if rel is not None:
                    @pl.when(cid == c)
                    def _(ri=_RIDX[rel]):
                        writeback(ri, seg_refs)

        if with_counts:
            zero_slices()
            plsc.subcore_barrier()
            for rel0, rel1 in _SLOTS:
                for c, rel in ((0, rel0), (1, rel1)):
                    if rel is not None:
                        @pl.when(cid == c)
                        def _(ri=_RIDX[rel]):
                            pltpu.sync_copy(oh_ref.at[ri], rows1)
                            count_loop(ri)
            plsc.subcore_barrier()
            for c in range(NC):
                @pl.when(cid == c)
                def _(c=c):
                    writeback(c, cnt_refs)

    return agg


_AGG_L1 = _build_agg(with_counts=True)
_AGG_L2 = _build_agg(with_counts=False)

_TCB = 1000


def _build_tc(n_rel, final):
    grid = (N // _TCB,)
    in_specs = [pl.BlockSpec((_TCB, D), lambda i: (i, 0))]
    for _ in range(n_rel):
        in_specs += [pl.BlockSpec((_TCB, D), lambda i: (i, 0)),
                     pl.BlockSpec((_TCB, 1), lambda i: (i, 0)),
                     pl.BlockSpec((D, D), lambda i: (0, 0))]
    in_specs += [pl.BlockSpec((D, D), lambda i: (0, 0)),
                 pl.BlockSpec((1, D), lambda i: (0, 0))]
    if final:
        in_specs += [pl.BlockSpec((D, 1), lambda i: (0, 0)),
                     pl.BlockSpec((1, 1), lambda i: (0, 0)),
                     pl.BlockSpec((1, 1), lambda i: (0, 0))]
        out_spec = pl.BlockSpec((_TCB, 1), lambda i: (i, 0))
        out_shape = jax.ShapeDtypeStruct((N, 1), jnp.float32)
    else:
        out_spec = pl.BlockSpec((_TCB, D), lambda i: (i, 0))
        out_shape = jax.ShapeDtypeStruct((N, D), jnp.float32)

    def body(*refs):
        x_ref = refs[0]
        k = 1
        acc = None
        for _ in range(n_rel):
            seg_ref, cnt_ref, w_ref = refs[k], refs[k + 1], refs[k + 2]
            k += 3
            recip = 1.0 / jnp.maximum(cnt_ref[...], 1.0)
            m = jnp.dot(seg_ref[...] * recip, w_ref[...],
                        preferred_element_type=jnp.float32)
            acc = m if acc is None else acc + m
        wr_ref, b_ref = refs[k], refs[k + 1]
        k += 2
        acc = acc + jnp.dot(x_ref[...], wr_ref[...],
                            preferred_element_type=jnp.float32) + b_ref[...]
        acc = jnp.maximum(acc, 0.0)
        if final:
            lw_ref, lb_ref, a_ref = refs[k], refs[k + 1], refs[k + 2]
            z = jnp.dot(acc, lw_ref[...],
                        preferred_element_type=jnp.float32) + lb_ref[...]
            acc = jnp.where(z >= 0, z, a_ref[...] * z)
        refs[-1][...] = acc

    return pl.pallas_call(body, grid=grid, in_specs=in_specs,
                          out_specs=out_spec, out_shape=out_shape)


_TC_CALLS = {}


def _tc_call(n_rel, final):
    key = (n_rel, final)
    if key not in _TC_CALLS:
        _TC_CALLS[key] = _build_tc(n_rel, final)
    return _TC_CALLS[key]


def _hetero_layer(xs, segs, cnts, layer_params, final_head=None):
    out = {}
    for d_t in _TYPES:
        rel_ids = [i for i, r in enumerate(_REL_DEFS) if r[2] == d_t]
        args = [xs[d_t]]
        wr_sum = None
        b_sum = None
        for ri in rel_ids:
            name = _REL_DEFS[ri][0]
            Wl, bl, Wr = layer_params[name]
            args += [segs[ri], cnts[ri], Wl.T]
            wr_sum = Wr if wr_sum is None else wr_sum + Wr
            b_sum = bl if b_sum is None else b_sum + bl
        args += [wr_sum.T, b_sum.reshape(1, D)]
        fin = final_head is not None and d_t in ('gw', 'sw')
        if fin:
            lin_W, lin_b, a = final_head
            args += [lin_W.T, lin_b.reshape(1, 1), a.reshape(1, 1)]
        out[d_t] = _tc_call(len(rel_ids), fin)(*args)
    return out


def kernel(x_pfas_sites, x_gw_wells, x_sw_stations,
           ei_ps_gw, ei_gw_ps, ei_ps_sw, ei_sw_ps, ei_sw_gw, ei_gw_sw,
           ei_gw_gw, ei_gw_self, ei_sw_self, ei_ps_self, params):
    xs = {'ps': x_pfas_sites, 'gw': x_gw_wells, 'sw': x_sw_stations}
    eis = {'ps_gw': ei_ps_gw, 'gw_ps': ei_gw_ps, 'ps_sw': ei_ps_sw,
           'sw_ps': ei_sw_ps, 'sw_gw': ei_sw_gw, 'gw_sw': ei_gw_sw,
           'gw_gw': ei_gw_gw, 'gw_self': ei_gw_self, 'sw_self': ei_sw_self,
           'ps_self': ei_ps_self}

    ei_pads = []
    for r in _REL_DEFS:
        ei = eis[r[0]].astype(jnp.int32)
        pad = _padded(r[3]) - r[3] + _OVERSHOOT
        padcols = jnp.stack([jnp.zeros((pad,), jnp.int32),
                             jnp.full((pad,), N, jnp.int32)])
        ei_pads.append(jnp.concatenate([ei, padcols], axis=1))
    zeros2d = jnp.zeros((N, D), jnp.float32)
    onehot = jnp.tile(jnp.eye(_NREL, D, dtype=jnp.float32)[:, None, :],
                      (1, _CHUNK, 1))

    out1 = _AGG_L1(xs['ps'], xs['gw'], xs['sw'], zeros2d, onehot,
                   *ei_pads)
    segs1 = list(out1[:_NREL])
    cnt_img = out1[_NREL:]
    cnts = [cnt_img[_CORE_OF[r[0]]][:, ri:ri + 1]
            for ri, r in enumerate(_REL_DEFS)]

    h1 = _hetero_layer(xs, segs1, cnts, params['conv1'])

    out2 = _AGG_L2(h1['ps'], h1['gw'], h1['sw'], zeros2d, onehot,
                   *ei_pads)
    segs2 = list(out2)

    head = (params['lin_W'], params['lin_b'], params['prelu_a'])
    h2 = _hetero_layer(h1, segs2, cnts, params['conv2'], final_head=head)

    return (h2['gw'], h2['sw'], h2['ps'])
